# Initial kernel scaffold; baseline (speedup 1.0000x reference)
import functools

import jax
import jax.numpy as jnp
from jax import lax
from jax.experimental import pallas as pl
from jax.experimental.pallas import tpu as pltpu
from jax.experimental.pallas import tpu_sc as plsc

N = 10000
E = 320000
F = 128

NS = 16
EPT = E // NS
CHUNK = 80
NCHUNK = EPT // CHUNK
RPT = N // NS
ZROWS = 125
VPR = F // 16


def _sc_agg_body(src1, dst1, ew1, src2, dst2, ew2, x,
                 out1, out2,
                 acc, rows, srcv, dstv, ewv, zbuf, gsem):
    cid = lax.axis_index("c")
    sid = lax.axis_index("s")

    def zrow(r, carry):
        for v in range(VPR):
            zbuf[r, pl.ds(v * 16, 16)] = jnp.zeros((16,), jnp.float32)
        return carry
    lax.fori_loop(0, ZROWS, zrow, 0)
    for k in range(RPT // ZROWS):
        pltpu.sync_copy(zbuf, acc.at[pl.ds(sid * RPT + k * ZROWS, ZROWS)])
    plsc.subcore_barrier()

    def process(srcE, dstE, ewE):
        base0 = sid * EPT

        def chunk_body(g, carry):
            off = base0 + g * CHUNK
            pltpu.sync_copy(srcE.at[pl.ds(off, CHUNK)], srcv.at[0])
            pltpu.sync_copy(dstE.at[pl.ds(off, CHUNK)], dstv.at[0])
            pltpu.sync_copy(ewE.at[pl.ds(off, CHUNK)], ewv.at[0])
            pltpu.async_copy(x.at[srcv.at[0]], rows.at[0], gsem).wait()

            def scale(e, c2):
                w = ewv[0, e]
                for v in range(VPR):
                    rows[0, e, pl.ds(v * 16, 16)] = (
                        rows[0, e, pl.ds(v * 16, 16)] * w)
                return c2
            lax.fori_loop(0, CHUNK, scale, 0)

            pltpu.sync_copy(rows.at[0], acc.at[dstv.at[0]], add=True)
            return carry
        lax.fori_loop(0, NCHUNK, chunk_body, 0)

    @pl.when(cid == 0)
    def _():
        process(src1, dst1, ew1)

    @pl.when(cid == 1)
    def _():
        process(src2, dst2, ew2)

    plsc.subcore_barrier()

    @pl.when(cid == 0)
    def _():
        for k in range(RPT // ZROWS):
            r0 = sid * RPT + k * ZROWS
            pltpu.sync_copy(acc.at[pl.ds(r0, ZROWS)], out1.at[pl.ds(r0, ZROWS)])

    @pl.when(cid == 1)
    def _():
        for k in range(RPT // ZROWS):
            r0 = sid * RPT + k * ZROWS
            pltpu.sync_copy(acc.at[pl.ds(r0, ZROWS)], out2.at[pl.ds(r0, ZROWS)])


_sc_agg = functools.partial(
    pl.kernel,
    out_type=[jax.ShapeDtypeStruct((N, F), jnp.float32),
              jax.ShapeDtypeStruct((N, F), jnp.float32)],
    mesh=plsc.VectorSubcoreMesh(core_axis_name="c", subcore_axis_name="s"),
    scratch_types=[
        pltpu.VMEM_SHARED((N, F), jnp.float32),
        pltpu.VMEM((1, CHUNK, F), jnp.float32),
        pltpu.VMEM((1, CHUNK), jnp.int32),
        pltpu.VMEM((1, CHUNK), jnp.int32),
        pltpu.VMEM((1, CHUNK), jnp.float32),
        pltpu.VMEM((ZROWS, F), jnp.float32),
        pltpu.SemaphoreType.DMA,
    ],
)(_sc_agg_body)


ROWB = 2000
GRID = N // ROWB


def _dense_body(last, x_ref, a1_ref, a2_ref, w0_ref, wa_ref, wb_ref, b_ref,
                o_ref):
    hi = lax.Precision.HIGHEST
    acc = lax.dot_general(x_ref[...], w0_ref[...],
                          (((1,), (1,)), ((), ())),
                          precision=hi, preferred_element_type=jnp.float32)
    acc += jnp.dot(a1_ref[...], wa_ref[...], precision=hi,
                   preferred_element_type=jnp.float32)
    acc += jnp.dot(a2_ref[...], wb_ref[...], precision=hi,
                   preferred_element_type=jnp.float32)
    acc += jnp.sum(b_ref[...], axis=0, keepdims=True)
    if last:
        m = jnp.max(acc, axis=1, keepdims=True)
        s = acc - m
        lse = jnp.log(jnp.sum(jnp.exp(s), axis=1, keepdims=True))
        acc = s - lse
    o_ref[...] = acc


def _dense_block(x, a1, a2, lnW, Wa, Wb, bstack, last):
    row_spec = pl.BlockSpec((ROWB, F), lambda i: (i, 0))
    full_spec = pl.BlockSpec((F, F), lambda i: (0, 0))
    return pl.pallas_call(
        functools.partial(_dense_body, last),
        grid=(GRID,),
        in_specs=[row_spec, row_spec, row_spec,
                  full_spec, full_spec, full_spec,
                  pl.BlockSpec((3, F), lambda i: (0, 0))],
        out_specs=row_spec,
        out_shape=jax.ShapeDtypeStruct((N, F), jnp.float32),
    )(x, a1, a2, lnW, Wa, Wb, bstack)


def kernel(features, edge_index, edge_index2, edge_weight, edge_weight2,
           lnW1, lnb1, cW1a, cb1a, cW1b, cb1b,
           lnW2, lnb2, cW2a, cb2a, cW2b, cb2b,
           lnW3, lnb3, cW3a, cb3a, cW3b, cb3b):
    src1, dst1 = edge_index[0], edge_index[1]
    src2, dst2 = edge_index2[0], edge_index2[1]

    blocks = [
        (lnW1, cW1a, cW1b, jnp.stack([lnb1, cb1a, cb1b])),
        (lnW2, cW2a, cW2b, jnp.stack([lnb2, cb2a, cb2b])),
        (lnW3, cW3a, cW3b, jnp.stack([lnb3, cb3a, cb3b])),
    ]

    x = features
    for b, (lnW, Wa, Wb, bstack) in enumerate(blocks):
        a1, a2 = _sc_agg(src1, dst1, edge_weight, src2, dst2, edge_weight2, x)
        x = _dense_block(x, a1, a2, lnW, Wa, Wb, bstack, last=(b == 2))
    return x


# speedup vs baseline: 3.0994x; 3.0994x over previous
import functools

import jax
import jax.numpy as jnp
from jax import lax
from jax.experimental import pallas as pl
from jax.experimental.pallas import tpu as pltpu
from jax.experimental.pallas import tpu_sc as plsc

N = 10000
E = 320000
F = 128

NS = 16
EPT = E // NS
CHUNK = 80
NCHUNK = EPT // CHUNK
ROWT = 640
WCH = 80
VPR = F // 16


def _sc_agg_body(src1, dst1, ew1, src2, dst2, ew2, x,
                 out1, out2,
                 acc, rows, srcv, dstv, ewv, zbuf, gsem):
    cid = lax.axis_index("c")
    sid = lax.axis_index("s")

    nch = jnp.where(sid == NS - 1, (N - (NS - 1) * ROWT) // WCH, ROWT // WCH)

    def zrow(r, carry):
        for v in range(VPR):
            zbuf[r, pl.ds(v * 16, 16)] = jnp.zeros((16,), jnp.float32)
        return carry
    lax.fori_loop(0, WCH, zrow, 0)

    def zcopy(k, carry):
        r0 = pl.multiple_of(sid * ROWT + k * WCH, 8)
        pltpu.sync_copy(zbuf, acc.at[pl.ds(r0, WCH)])
        return carry
    lax.fori_loop(0, nch, zcopy, 0)
    plsc.subcore_barrier()

    def process(srcE, dstE, ewE):
        base0 = sid * EPT

        def chunk_body(g, carry):
            off = pl.multiple_of(base0 + g * CHUNK, 8)
            pltpu.sync_copy(srcE.at[pl.ds(off, CHUNK)], srcv.at[0])
            pltpu.sync_copy(dstE.at[pl.ds(off, CHUNK)], dstv.at[0])
            pltpu.sync_copy(ewE.at[pl.ds(off, CHUNK)], ewv.at[0])
            pltpu.async_copy(x.at[srcv.at[0]], rows.at[0], gsem).wait()

            def scale(q, c2):
                wvec = ewv[0, pl.ds(q * 16, 16)]
                for j in range(16):
                    w = wvec[j]
                    e = q * 16 + j
                    for v in range(VPR):
                        rows[0, e, pl.ds(v * 16, 16)] = (
                            rows[0, e, pl.ds(v * 16, 16)] * w)
                return c2
            lax.fori_loop(0, CHUNK // 16, scale, 0)

            pltpu.sync_copy(rows.at[0], acc.at[dstv.at[0]], add=True)
            return carry
        lax.fori_loop(0, NCHUNK, chunk_body, 0)

    @pl.when(cid == 0)
    def _():
        process(src1, dst1, ew1)

    @pl.when(cid == 1)
    def _():
        process(src2, dst2, ew2)

    plsc.subcore_barrier()

    def writeout(out_ref):
        def wcopy(k, carry):
            r0 = pl.multiple_of(sid * ROWT + k * WCH, 8)
            pltpu.sync_copy(acc.at[pl.ds(r0, WCH)], out_ref.at[pl.ds(r0, WCH)])
            return carry
        lax.fori_loop(0, nch, wcopy, 0)

    @pl.when(cid == 0)
    def _():
        writeout(out1)

    @pl.when(cid == 1)
    def _():
        writeout(out2)


_sc_agg = functools.partial(
    pl.kernel,
    out_type=[jax.ShapeDtypeStruct((N, F), jnp.float32),
              jax.ShapeDtypeStruct((N, F), jnp.float32)],
    mesh=plsc.VectorSubcoreMesh(core_axis_name="c", subcore_axis_name="s"),
    scratch_types=[
        pltpu.VMEM_SHARED((N, F), jnp.float32),
        pltpu.VMEM((1, CHUNK, F), jnp.float32),
        pltpu.VMEM((1, CHUNK), jnp.int32),
        pltpu.VMEM((1, CHUNK), jnp.int32),
        pltpu.VMEM((1, CHUNK), jnp.float32),
        pltpu.VMEM((WCH, F), jnp.float32),
        pltpu.SemaphoreType.DMA,
    ],
)(_sc_agg_body)


ROWB = 2000
GRID = N // ROWB


def _dense_body(last, x_ref, a1_ref, a2_ref, w0_ref, wa_ref, wb_ref, b_ref,
                o_ref):
    hi = lax.Precision.HIGHEST
    acc = lax.dot_general(x_ref[...], w0_ref[...],
                          (((1,), (1,)), ((), ())),
                          precision=hi, preferred_element_type=jnp.float32)
    acc += jnp.dot(a1_ref[...], wa_ref[...], precision=hi,
                   preferred_element_type=jnp.float32)
    acc += jnp.dot(a2_ref[...], wb_ref[...], precision=hi,
                   preferred_element_type=jnp.float32)
    acc += jnp.sum(b_ref[...], axis=0, keepdims=True)
    if last:
        m = jnp.max(acc, axis=1, keepdims=True)
        s = acc - m
        lse = jnp.log(jnp.sum(jnp.exp(s), axis=1, keepdims=True))
        acc = s - lse
    o_ref[...] = acc


def _dense_block(x, a1, a2, lnW, Wa, Wb, bstack, last):
    row_spec = pl.BlockSpec((ROWB, F), lambda i: (i, 0))
    full_spec = pl.BlockSpec((F, F), lambda i: (0, 0))
    return pl.pallas_call(
        functools.partial(_dense_body, last),
        grid=(GRID,),
        in_specs=[row_spec, row_spec, row_spec,
                  full_spec, full_spec, full_spec,
                  pl.BlockSpec((3, F), lambda i: (0, 0))],
        out_specs=row_spec,
        out_shape=jax.ShapeDtypeStruct((N, F), jnp.float32),
    )(x, a1, a2, lnW, Wa, Wb, bstack)


def kernel(features, edge_index, edge_index2, edge_weight, edge_weight2,
           lnW1, lnb1, cW1a, cb1a, cW1b, cb1b,
           lnW2, lnb2, cW2a, cb2a, cW2b, cb2b,
           lnW3, lnb3, cW3a, cb3a, cW3b, cb3b):
    src1, dst1 = edge_index[0], edge_index[1]
    src2, dst2 = edge_index2[0], edge_index2[1]

    blocks = [
        (lnW1, cW1a, cW1b, jnp.stack([lnb1, cb1a, cb1b])),
        (lnW2, cW2a, cW2b, jnp.stack([lnb2, cb2a, cb2b])),
        (lnW3, cW3a, cW3b, jnp.stack([lnb3, cb3a, cb3b])),
    ]

    x = features
    for b, (lnW, Wa, Wb, bstack) in enumerate(blocks):
        a1, a2 = _sc_agg(src1, dst1, edge_weight, src2, dst2, edge_weight2, x)
        x = _dense_block(x, a1, a2, lnW, Wa, Wb, bstack, last=(b == 2))
    return x
